# baseline (speedup 1.0000x reference)
import numpy as np

import jax
import jax.numpy as jnp
from jax import lax
from jax.experimental import pallas as pl
from jax.experimental.pallas import tpu as pltpu
from jax.experimental.pallas import tpu_sc as plsc

B = 4096
L_SEQ = 200
EMB = 64
EMBH = EMB // 2
HIDDEN = 768
VOCAB_N = 100000

NC = 2
NS = 16
NW = NC * NS
EPW = B // NW
C0 = 104
C1 = L_SEQ - C0

UNROLL = 8
NBUF = 4

_PERM = np.concatenate([
    np.arange(0, 32, 2), np.arange(1, 32, 2),
    np.arange(32, 64, 2), np.arange(33, 64, 2),
])


def _sc_pool_body(ids_hbm, table_hbm, out_hbm, idx_v, rows_v, out_v, sem):
    wid = lax.axis_index("s") * NC + lax.axis_index("c")
    pltpu.sync_copy(ids_hbm.at[wid], idx_v)

    inv_l = jnp.full((16,), 1.0 / L_SEQ, dtype=jnp.float32)
    himask = jnp.full((16,), -65536, dtype=jnp.int32)

    def issue(b, buf):
        pltpu.async_copy(
            table_hbm.at[idx_v.at[b, pl.ds(0, C0)]],
            rows_v.at[buf, pl.ds(0, C0)], sem.at[buf])
        pltpu.async_copy(
            table_hbm.at[idx_v.at[b, pl.ds(C0, C1)]],
            rows_v.at[buf, pl.ds(C0, C1)], sem.at[buf])

    for p in range(NBUF - 1):
        issue(p, p)

    def per_row(b, _):
        buf = lax.rem(b, NBUF)

        @pl.when(b < EPW - (NBUF - 1))
        def _():
            issue(b + NBUF - 1, lax.rem(b + NBUF - 1, NBUF))

        pltpu.make_async_copy(
            table_hbm.at[pl.ds(0, L_SEQ)], rows_v.at[buf], sem.at[buf]
        ).wait()

        def acc_body(t, carry):
            a0, a1 = carry
            base = t * UNROLL
            for u in range(UNROLL):
                j = base + u
                v0 = plsc.bitcast(rows_v[buf, j, pl.ds(0, 32)], jnp.int32)
                a0 = a0 + plsc.bitcast(v0 << 16, jnp.float32)
                a1 = a1 + plsc.bitcast(v0 & himask, jnp.float32)
            return a0, a1

        z = jnp.zeros((16,), jnp.float32)
        a0, a1 = lax.fori_loop(0, L_SEQ // UNROLL, acc_body, (z, z))
        out_v[b, pl.ds(0, 16)] = a0 * inv_l
        out_v[b, pl.ds(16, 16)] = a1 * inv_l
        return 0

    lax.fori_loop(0, EPW, per_row, 0)
    pltpu.sync_copy(out_v, out_hbm.at[wid])


def _sc_pool(ids3, tablehalf):
    mesh = plsc.VectorSubcoreMesh(
        core_axis_name="c", subcore_axis_name="s", num_cores=NC,
        num_subcores=NS)
    k = pl.kernel(
        _sc_pool_body,
        out_type=jax.ShapeDtypeStruct((NW, EPW, EMBH), jnp.float32),
        mesh=mesh,
        scratch_types=[
            pltpu.VMEM((EPW, L_SEQ), jnp.int32),
            pltpu.VMEM((NBUF, L_SEQ, EMBH), jnp.bfloat16),
            pltpu.VMEM((EPW, EMBH), jnp.float32),
            pltpu.SemaphoreType.DMA((NBUF,)),
        ],
        compiler_params=pltpu.CompilerParams(
            use_tc_tiling_on_sc=False, needs_layout_passes=False),
    )
    return k(ids3, tablehalf)


def _mlp_body(x_ref, w1_ref, b1_ref, w2t_ref, b2_ref, out_ref):
    x = x_ref[...]
    h = jnp.dot(x, w1_ref[...], preferred_element_type=jnp.float32)
    h = jnp.maximum(h + b1_ref[...], 0.0)
    o = jnp.sum(h * w2t_ref[...], axis=1) + b2_ref[0, 0]
    out_ref[...] = o[None, :]


def _mlp(pooled, W1p, b1, W2, b2):
    nblk = 8
    bblk = B // nblk
    out = pl.pallas_call(
        _mlp_body,
        grid=(nblk,),
        in_specs=[
            pl.BlockSpec((bblk, EMB), lambda i: (i, 0)),
            pl.BlockSpec((EMB, HIDDEN), lambda i: (0, 0)),
            pl.BlockSpec((1, HIDDEN), lambda i: (0, 0)),
            pl.BlockSpec((1, HIDDEN), lambda i: (0, 0)),
            pl.BlockSpec((1, 1), lambda i: (0, 0)),
        ],
        out_specs=pl.BlockSpec((1, bblk), lambda i: (0, i)),
        out_shape=jax.ShapeDtypeStruct((1, B), jnp.float32),
    )(pooled, W1p, b1.reshape(1, HIDDEN), W2.reshape(1, HIDDEN),
      b2.reshape(1, 1))
    return out.reshape(B)


def kernel(ids, table, W1, b1, W2, b2):
    ids3 = ids.astype(jnp.int32).reshape(NW, EPW, L_SEQ)
    ta = table[:, :EMBH].astype(jnp.bfloat16)
    tb = table[:, EMBH:].astype(jnp.bfloat16)
    pa = _sc_pool(ids3, ta)
    pb = _sc_pool(ids3, tb)
    pooled = jnp.concatenate([pa, pb], axis=-1).reshape(B, EMB)
    W1p = jnp.take(W1, _PERM, axis=0)
    return _mlp(pooled, W1p, b1, W2, b2)


# speedup vs baseline: 1.4977x; 1.4977x over previous
import numpy as np

import jax
import jax.numpy as jnp
from jax import lax
from jax.experimental import pallas as pl
from jax.experimental.pallas import tpu as pltpu
from jax.experimental.pallas import tpu_sc as plsc

B = 4096
L_SEQ = 200
EMB = 64
HIDDEN = 768
VOCAB_N = 100000

NC = 2
NS = 16
NW = NC * NS
EPW = B // NW
C0 = 104
C1 = L_SEQ - C0

UNROLL = 8
NBUF = 8

_PERM = np.concatenate([
    np.arange(0, 32, 2), np.arange(1, 32, 2),
    np.arange(32, 64, 2), np.arange(33, 64, 2),
])


def _sc_pool_body(ids_hbm, table_hbm, out_hbm, idx_v, rows_v, out_v, sem):
    wid = lax.axis_index("s") * NC + lax.axis_index("c")
    pltpu.sync_copy(ids_hbm.at[wid], idx_v)

    inv_l = jnp.full((16,), 1.0 / L_SEQ, dtype=jnp.float32)
    himask = jnp.full((16,), -65536, dtype=jnp.int32)

    def issue(b, buf):
        pltpu.async_copy(
            table_hbm.at[idx_v.at[b, pl.ds(0, C0)]],
            rows_v.at[buf, pl.ds(0, C0)], sem.at[buf])
        pltpu.async_copy(
            table_hbm.at[idx_v.at[b, pl.ds(C0, C1)]],
            rows_v.at[buf, pl.ds(C0, C1)], sem.at[buf])

    for p in range(NBUF - 1):
        issue(p, p)

    def per_row(b, _):
        buf = lax.rem(b, NBUF)

        @pl.when(b < EPW - (NBUF - 1))
        def _():
            issue(b + NBUF - 1, lax.rem(b + NBUF - 1, NBUF))

        pltpu.make_async_copy(
            table_hbm.at[pl.ds(0, L_SEQ)], rows_v.at[buf], sem.at[buf]
        ).wait()

        def acc_body(t, carry):
            a0, a1, a2, a3 = carry
            base = t * UNROLL
            for u in range(UNROLL):
                j = base + u
                v0 = plsc.bitcast(rows_v[buf, j, pl.ds(0, 32)], jnp.int32)
                v1 = plsc.bitcast(rows_v[buf, j, pl.ds(32, 32)], jnp.int32)
                a0 = a0 + plsc.bitcast(v0 << 16, jnp.float32)
                a1 = a1 + plsc.bitcast(v0 & himask, jnp.float32)
                a2 = a2 + plsc.bitcast(v1 << 16, jnp.float32)
                a3 = a3 + plsc.bitcast(v1 & himask, jnp.float32)
            return a0, a1, a2, a3

        z = jnp.zeros((16,), jnp.float32)
        a0, a1, a2, a3 = lax.fori_loop(
            0, L_SEQ // UNROLL, acc_body, (z, z, z, z))
        out_v[b, pl.ds(0, 16)] = a0 * inv_l
        out_v[b, pl.ds(16, 16)] = a1 * inv_l
        out_v[b, pl.ds(32, 16)] = a2 * inv_l
        out_v[b, pl.ds(48, 16)] = a3 * inv_l
        return 0

    lax.fori_loop(0, EPW, per_row, 0)
    pltpu.sync_copy(out_v, out_hbm.at[wid])


def _sc_pool(ids3, tableb):
    mesh = plsc.VectorSubcoreMesh(
        core_axis_name="c", subcore_axis_name="s", num_cores=NC,
        num_subcores=NS)
    k = pl.kernel(
        _sc_pool_body,
        out_type=jax.ShapeDtypeStruct((NW, EPW, EMB), jnp.float32),
        mesh=mesh,
        scratch_types=[
            pltpu.VMEM((EPW, L_SEQ), jnp.int32),
            pltpu.VMEM((NBUF, L_SEQ, EMB), jnp.bfloat16),
            pltpu.VMEM((EPW, EMB), jnp.float32),
            pltpu.SemaphoreType.DMA((NBUF,)),
        ],
        compiler_params=pltpu.CompilerParams(
            use_tc_tiling_on_sc=False, needs_layout_passes=False),
    )
    return k(ids3, tableb)


def _mlp_body(x_ref, w1_ref, b1_ref, w2t_ref, b2_ref, out_ref):
    x = x_ref[...].astype(jnp.bfloat16)
    h = jnp.dot(x, w1_ref[...], preferred_element_type=jnp.float32)
    h = jnp.maximum(h + b1_ref[...], 0.0)
    o = jnp.sum(h * w2t_ref[...], axis=1) + b2_ref[0, 0]
    out_ref[...] = o[None, :]


def _mlp(pooled, W1p, b1, W2, b2):
    nblk = 8
    bblk = B // nblk
    out = pl.pallas_call(
        _mlp_body,
        grid=(nblk,),
        in_specs=[
            pl.BlockSpec((bblk, EMB), lambda i: (i, 0)),
            pl.BlockSpec((EMB, HIDDEN), lambda i: (0, 0)),
            pl.BlockSpec((1, HIDDEN), lambda i: (0, 0)),
            pl.BlockSpec((1, HIDDEN), lambda i: (0, 0)),
            pl.BlockSpec((1, 1), lambda i: (0, 0)),
        ],
        out_specs=pl.BlockSpec((1, bblk), lambda i: (0, i)),
        out_shape=jax.ShapeDtypeStruct((1, B), jnp.float32),
    )(pooled, W1p, b1.reshape(1, HIDDEN), W2.reshape(1, HIDDEN),
      b2.reshape(1, 1))
    return out.reshape(B)


def kernel(ids, table, W1, b1, W2, b2):
    ids3 = ids.astype(jnp.int32).reshape(NW, EPW, L_SEQ)
    tableb = table.astype(jnp.bfloat16)
    pooled = _sc_pool(ids3, tableb).reshape(B, EMB)
    W1p = jnp.take(W1, _PERM, axis=0).astype(jnp.bfloat16)
    return _mlp(pooled, W1p, b1, W2, b2)
